# Initial kernel scaffold; baseline (speedup 1.0000x reference)
import jax
import jax.numpy as jnp
from jax.experimental import pallas as pl


def kernel(item_history, cate_history, targetitem, targetcate, item_lookup, cate_lookup, gamma, beta, W1, b1, a1, W2, b2, a2, W3, b3):
    raise NotImplementedError("write your pallas kernel here")


# speedup vs baseline: 8.0495x; 8.0495x over previous
import functools

import jax
import jax.numpy as jnp
from jax import lax
from jax.experimental import pallas as pl
from jax.experimental.pallas import tpu as pltpu
from jax.experimental.pallas import tpu_sc as plsc

NC = 2
NS = 16
NW = NC * NS
LANES = 16

B = 16384
L = 200
D = 64
HALF = L // 2

ROWS_PER_W = B // NW
G = 4
TGT_CHUNK = 128


def _sc_gather_kernel(item_hist_hbm, cate_hist_hbm, titem_hbm, tcate_hbm,
                      item_tab_hbm, cate_tab_hbm,
                      ti_out, tc_out, ihs_out, chs_out,
                      ih_idx, ch_idx, ibuf, cbuf, obuf_i, obuf_c,
                      tidx, trows, sem):
  cid = lax.axis_index("c")
  sid = lax.axis_index("s")
  wid = sid * NC + cid
  base = wid * ROWS_PER_W

  def tgt_loop(tab_hbm, src_idx_hbm, dst_hbm):
    def body(j, _):
      off = base + j * TGT_CHUNK
      pltpu.sync_copy(src_idx_hbm.at[pl.ds(off, TGT_CHUNK)], tidx)
      pltpu.async_copy(tab_hbm.at[tidx], trows, sem).wait()
      pltpu.sync_copy(trows, dst_hbm.at[pl.ds(off, TGT_CHUNK)])
      return _
    return lax.fori_loop(0, ROWS_PER_W // TGT_CHUNK, body, 0)

  tgt_loop(item_tab_hbm, titem_hbm, ti_out)
  tgt_loop(cate_tab_hbm, tcate_hbm, tc_out)

  def group_body(grp, _):
    b0 = base + grp * G
    pltpu.sync_copy(item_hist_hbm.at[pl.ds(b0, G)], ih_idx)
    pltpu.sync_copy(cate_hist_hbm.at[pl.ds(b0, G)], ch_idx)
    cps = []
    for g in range(G):
      for h in range(2):
        cps.append(pltpu.async_copy(
            item_tab_hbm.at[ih_idx.at[g, h]], ibuf.at[g, h], sem))
        cps.append(pltpu.async_copy(
            cate_tab_hbm.at[ch_idx.at[g, h]], cbuf.at[g, h], sem))
    for c in cps:
      c.wait()

    for g in range(G):
      def red_body(l, accs):
        ai0, ai1, ai2, ai3, ac0, ac1, ac2, ac3 = accs
        h = l // HALF
        r = l - h * HALF
        ai0 = ai0 + ibuf[g, h, r, pl.ds(0, LANES)]
        ai1 = ai1 + ibuf[g, h, r, pl.ds(16, LANES)]
        ai2 = ai2 + ibuf[g, h, r, pl.ds(32, LANES)]
        ai3 = ai3 + ibuf[g, h, r, pl.ds(48, LANES)]
        ac0 = ac0 + cbuf[g, h, r, pl.ds(0, LANES)]
        ac1 = ac1 + cbuf[g, h, r, pl.ds(16, LANES)]
        ac2 = ac2 + cbuf[g, h, r, pl.ds(32, LANES)]
        ac3 = ac3 + cbuf[g, h, r, pl.ds(48, LANES)]
        return ai0, ai1, ai2, ai3, ac0, ac1, ac2, ac3

      z = jnp.zeros((LANES,), jnp.float32)
      accs = lax.fori_loop(0, L, red_body, (z, z, z, z, z, z, z, z))
      for d in range(4):
        obuf_i[g, pl.ds(d * LANES, LANES)] = accs[d]
        obuf_c[g, pl.ds(d * LANES, LANES)] = accs[4 + d]

    pltpu.sync_copy(obuf_i, ihs_out.at[pl.ds(b0, G)])
    pltpu.sync_copy(obuf_c, chs_out.at[pl.ds(b0, G)])
    return _

  lax.fori_loop(0, ROWS_PER_W // G, group_body, 0)


def _sc_gather(item_hist, cate_hist, titem, tcate, item_tab, cate_tab):
  mesh = plsc.VectorSubcoreMesh(core_axis_name="c", subcore_axis_name="s",
                                num_cores=NC, num_subcores=NS)
  f32 = jnp.float32
  out_type = (
      jax.ShapeDtypeStruct((B, D), f32),
      jax.ShapeDtypeStruct((B, D), f32),
      jax.ShapeDtypeStruct((B, D), f32),
      jax.ShapeDtypeStruct((B, D), f32),
  )
  scratch = [
      pltpu.VMEM((G, 2, HALF), jnp.int32),
      pltpu.VMEM((G, 2, HALF), jnp.int32),
      pltpu.VMEM((G, 2, HALF, D), f32),
      pltpu.VMEM((G, 2, HALF, D), f32),
      pltpu.VMEM((G, D), f32),
      pltpu.VMEM((G, D), f32),
      pltpu.VMEM((TGT_CHUNK,), jnp.int32),
      pltpu.VMEM((TGT_CHUNK, D), f32),
      pltpu.SemaphoreType.DMA,
  ]
  fn = pl.kernel(_sc_gather_kernel, out_type=out_type, mesh=mesh,
                 scratch_types=scratch,
                 compiler_params=pltpu.CompilerParams(
                     use_tc_tiling_on_sc=False))
  return fn(item_hist, cate_hist, titem, tcate, item_tab, cate_tab)


BS = 1024


def _mlp_kernel(ti, tc, ihs, chs, w1, b1, a1, w2, b2, a2, w3, b3, out):
  x = jnp.concatenate([ti[...], tc[...], ihs[...], chs[...]], axis=1)
  h1 = jnp.dot(x, w1[...], preferred_element_type=jnp.float32) + b1[...]
  h1 = jnp.maximum(h1, 0.0) + a1[...] * jnp.minimum(h1, 0.0)
  h2 = jnp.dot(h1, w2[...], preferred_element_type=jnp.float32) + b2[...]
  h2 = jnp.maximum(h2, 0.0) + a2[...] * jnp.minimum(h2, 0.0)
  z = jnp.dot(h2, w3[...], preferred_element_type=jnp.float32) + b3[...]
  m = jnp.max(z, axis=-1, keepdims=True)
  e = jnp.exp(z - m)
  out[...] = e / jnp.sum(e, axis=-1, keepdims=True) + 1e-8


def _mlp(ti, tc, ihs, chs, w1, b1, a1, w2, b2, a2, w3, b3):
  n1, n2 = w2.shape
  n3 = w3.shape[1]
  row = lambda i: (i, 0)
  full = lambda i: (0, 0)
  bspec = lambda: pl.BlockSpec((BS, D), row)
  wspec = lambda s: pl.BlockSpec(s, full)
  return pl.pallas_call(
      _mlp_kernel,
      grid=(B // BS,),
      in_specs=[
          bspec(), bspec(), bspec(), bspec(),
          wspec((4 * D, n1)), wspec((1, n1)), wspec((1, n1)),
          wspec((n1, n2)), wspec((1, n2)), wspec((1, n2)),
          wspec((n2, n3)), wspec((1, n3)),
      ],
      out_specs=pl.BlockSpec((BS, n3), row),
      out_shape=jax.ShapeDtypeStruct((B, n3), jnp.float32),
  )(ti, tc, ihs, chs, w1, b1, a1, w2, b2, a2, w3, b3)


def kernel(item_history, cate_history, targetitem, targetcate, item_lookup,
           cate_lookup, gamma, beta, W1, b1, a1, W2, b2, a2, W3, b3):
  ih = item_history.reshape(B, 2, HALF)
  ch = cate_history.reshape(B, 2, HALF)
  ti, tc, ihs, chs = _sc_gather(ih, ch, targetitem, targetcate,
                                item_lookup, cate_lookup)
  scale = gamma * (1.0 / jnp.sqrt(1.0 + 1e-3))
  w1 = W1 * scale[:, None]
  b1e = (b1 + beta @ W1).reshape(1, -1)
  out = _mlp(ti, tc, ihs, chs, w1, b1e, a1.reshape(1, -1),
             W2, b2.reshape(1, -1), a2.reshape(1, -1),
             W3, b3.reshape(1, -1))
  return out


# speedup vs baseline: 14.7634x; 1.8341x over previous
import jax
import jax.numpy as jnp
from jax import lax
from jax.experimental import pallas as pl
from jax.experimental.pallas import tpu as pltpu
from jax.experimental.pallas import tpu_sc as plsc

NC = 2
NS = 16
NW = NC * NS
LANES = 16

B = 16384
L = 200
D = 64
HALF = L // 2
NCATE = 1024

ROWS_PER_W = B // NW
G = 4
NGRP = ROWS_PER_W // G
TGT_CHUNK = 32
NTC = ROWS_PER_W // TGT_CHUNK


def _sc_gather_kernel(item_hist_hbm, cate_hist_hbm, titem_hbm, tcate_hbm,
                      item_tab_hbm, cate_tab_hbm,
                      ti_out, tc_out, ihs_out, cnt_out,
                      ih_idx, ch_idx, ibuf, obuf_i, cbuf,
                      tidx_i, tidx_c, tbuf,
                      gsem, isem, ssem, tgsem, tssem):
  cid = lax.axis_index("c")
  sid = lax.axis_index("s")
  wid = sid * NC + cid
  base = wid * ROWS_PER_W

  pltpu.sync_copy(titem_hbm.at[pl.ds(base, ROWS_PER_W)], tidx_i)
  pltpu.sync_copy(tcate_hbm.at[pl.ds(base, ROWS_PER_W)], tidx_c)

  def tgt_gather(c):
    tab = item_tab_hbm if c < NTC else cate_tab_hbm
    idx = tidx_i if c < NTC else tidx_c
    off = (c % NTC) * TGT_CHUNK
    return pltpu.make_async_copy(
        tab.at[idx.at[pl.ds(off, TGT_CHUNK)]], tbuf.at[c % 4],
        tgsem.at[c % 4])

  def tgt_store(c):
    dst = ti_out if c < NTC else tc_out
    off = base + (c % NTC) * TGT_CHUNK
    return pltpu.make_async_copy(
        tbuf.at[c % 4], dst.at[pl.ds(off, TGT_CHUNK)], tssem.at[c % 4])

  tgt_gather(0).start()
  tgt_gather(1).start()
  for c in range(2 * NTC):
    tgt_gather(c).wait()
    if c >= 2:
      tgt_store(c - 2).wait()
    tgt_store(c).start()
    if c + 2 < 2 * NTC:
      tgt_gather(c + 2).start()
  tgt_store(2 * NTC - 2).wait()
  tgt_store(2 * NTC - 1).wait()

  ones = jnp.ones((LANES,), jnp.float32)
  lastmask = lax.iota(jnp.int32, LANES) >= (2 * LANES - (L % (2 * LANES)))
  zeros = jnp.zeros((LANES,), jnp.float32)

  def idx_copies(k, p):
    b0 = base + k * G
    return [
        pltpu.make_async_copy(
            item_hist_hbm.at[pl.ds(b0, G)], ih_idx.at[p], isem.at[p]),
        pltpu.make_async_copy(
            cate_hist_hbm.at[pl.ds(b0, G)], ch_idx.at[p], isem.at[p]),
    ]

  def gather_copies(p):
    cps = []
    for g in range(G):
      for h in range(2):
        cps.append(pltpu.make_async_copy(
            item_tab_hbm.at[ih_idx.at[p, g, h]], ibuf.at[p, g, h],
            gsem.at[p]))
    return cps

  def store_copies(k, p):
    b0 = base + k * G
    return [
        pltpu.make_async_copy(
            obuf_i.at[p], ihs_out.at[pl.ds(b0, G)], ssem.at[p]),
        pltpu.make_async_copy(
            cbuf.at[p], cnt_out.at[pl.ds(b0, G)], ssem.at[p]),
    ]

  for p in range(2):
    for c in idx_copies(p, p):
      c.start()
      c.wait()
    for c in gather_copies(p):
      c.start()

  def loop_body(jj, carry):
    for m in range(2):
      p = m
      k = 2 * jj + m
      for c in gather_copies(p):
        c.wait()
      @pl.when(jj >= 1)
      def _():
        for c in store_copies(k, p):
          c.wait()
      for g in range(G):
        for c in range(NCATE // LANES):
          cbuf[p, g, pl.ds(c * LANES, LANES)] = zeros
        gsplat = jnp.full((LANES,), g, jnp.int32)
        for c in range(L // LANES):
          idxv = ch_idx[p, g, pl.ds(c * LANES, LANES)]
          plsc.addupdate_scatter(cbuf.at[p], [gsplat, idxv], ones)
        idxv = ch_idx[p, g, pl.ds(L - LANES, LANES)]
        plsc.addupdate_scatter(cbuf.at[p], [gsplat, idxv], ones,
                               mask=lastmask)
      @pl.when(jj <= (NGRP // 2 - 2))
      def _():
        for c in idx_copies(k + 2, p):
          c.start()
      for g in range(G):
        def red_body(i, accs):
          a0, a1, a2, a3 = accs
          for h in range(2):
            for u in range(4):
              r = i * 4 + u
              a0 = a0 + ibuf[p, g, h, r, pl.ds(0, LANES)]
              a1 = a1 + ibuf[p, g, h, r, pl.ds(16, LANES)]
              a2 = a2 + ibuf[p, g, h, r, pl.ds(32, LANES)]
              a3 = a3 + ibuf[p, g, h, r, pl.ds(48, LANES)]
          return a0, a1, a2, a3

        accs = lax.fori_loop(0, HALF // 4, red_body, (zeros,) * 4)
        for d in range(4):
          obuf_i[p, g, pl.ds(d * LANES, LANES)] = accs[d]
      for c in store_copies(k, p):
        c.start()
      @pl.when(jj <= (NGRP // 2 - 2))
      def _():
        for c in idx_copies(k + 2, p):
          c.wait()
        for c in gather_copies(p):
          c.start()
    return carry

  lax.fori_loop(0, NGRP // 2, loop_body, 0)

  for p in range(2):
    for c in store_copies(NGRP - 2 + p, p):
      c.wait()


def _sc_gather(item_hist, cate_hist, titem, tcate, item_tab, cate_tab):
  mesh = plsc.VectorSubcoreMesh(core_axis_name="c", subcore_axis_name="s",
                                num_cores=NC, num_subcores=NS)
  f32 = jnp.float32
  i32 = jnp.int32
  out_type = (
      jax.ShapeDtypeStruct((B, D), f32),
      jax.ShapeDtypeStruct((B, D), f32),
      jax.ShapeDtypeStruct((B, D), f32),
      jax.ShapeDtypeStruct((B, NCATE), f32),
  )
  scratch = [
      pltpu.VMEM((2, G, 2, HALF), i32),
      pltpu.VMEM((2, G, L), i32),
      pltpu.VMEM((2, G, 2, HALF, D), f32),
      pltpu.VMEM((2, G, D), f32),
      pltpu.VMEM((2, G, NCATE), f32),
      pltpu.VMEM((ROWS_PER_W,), i32),
      pltpu.VMEM((ROWS_PER_W,), i32),
      pltpu.VMEM((4, TGT_CHUNK, D), f32),
      pltpu.SemaphoreType.DMA((2,)),
      pltpu.SemaphoreType.DMA((2,)),
      pltpu.SemaphoreType.DMA((2,)),
      pltpu.SemaphoreType.DMA((4,)),
      pltpu.SemaphoreType.DMA((4,)),
  ]
  fn = pl.kernel(_sc_gather_kernel, out_type=out_type, mesh=mesh,
                 scratch_types=scratch,
                 compiler_params=pltpu.CompilerParams(
                     use_tc_tiling_on_sc=False,
                     needs_layout_passes=False))
  return fn(item_hist, cate_hist, titem, tcate, item_tab, cate_tab)


BS = 1024


def _mlp_kernel(ti, tc, ihs, cnt, ctab, w1, b1, a1, w2, b2, a2, w3, b3, out):
  chs = jnp.dot(cnt[...], ctab[...], preferred_element_type=jnp.float32)
  x = jnp.concatenate([ti[...], tc[...], ihs[...], chs], axis=1)
  h1 = jnp.dot(x, w1[...], preferred_element_type=jnp.float32) + b1[...]
  h1 = jnp.maximum(h1, 0.0) + a1[...] * jnp.minimum(h1, 0.0)
  h2 = jnp.dot(h1, w2[...], preferred_element_type=jnp.float32) + b2[...]
  h2 = jnp.maximum(h2, 0.0) + a2[...] * jnp.minimum(h2, 0.0)
  z = jnp.dot(h2, w3[...], preferred_element_type=jnp.float32) + b3[...]
  m = jnp.max(z, axis=-1, keepdims=True)
  e = jnp.exp(z - m)
  out[...] = e / jnp.sum(e, axis=-1, keepdims=True) + 1e-8


def _mlp(ti, tc, ihs, cnt, ctab, w1, b1, a1, w2, b2, a2, w3, b3):
  n1, n2 = w2.shape
  n3 = w3.shape[1]
  row = lambda i: (i, 0)
  full = lambda i: (0, 0)
  bspec = lambda w: pl.BlockSpec((BS, w), row)
  wspec = lambda s: pl.BlockSpec(s, full)
  return pl.pallas_call(
      _mlp_kernel,
      grid=(B // BS,),
      in_specs=[
          bspec(D), bspec(D), bspec(D), bspec(NCATE),
          wspec((NCATE, D)),
          wspec((4 * D, n1)), wspec((1, n1)), wspec((1, n1)),
          wspec((n1, n2)), wspec((1, n2)), wspec((1, n2)),
          wspec((n2, n3)), wspec((1, n3)),
      ],
      out_specs=pl.BlockSpec((BS, n3), row),
      out_shape=jax.ShapeDtypeStruct((B, n3), jnp.float32),
  )(ti, tc, ihs, cnt, ctab, w1, b1, a1, w2, b2, a2, w3, b3)


def kernel(item_history, cate_history, targetitem, targetcate, item_lookup,
           cate_lookup, gamma, beta, W1, b1, a1, W2, b2, a2, W3, b3):
  ih = item_history.reshape(B, 2, HALF)
  ti, tc, ihs, cnt = _sc_gather(ih, cate_history, targetitem, targetcate,
                                item_lookup, cate_lookup)
  ctab = jnp.pad(cate_lookup, ((0, NCATE - cate_lookup.shape[0]), (0, 0)))
  scale = gamma * (1.0 / jnp.sqrt(1.0 + 1e-3))
  w1 = W1 * scale[:, None]
  b1e = (b1 + beta @ W1).reshape(1, -1)
  out = _mlp(ti, tc, ihs, cnt, ctab, w1, b1e, a1.reshape(1, -1),
             W2, b2.reshape(1, -1), a2.reshape(1, -1),
             W3, b3.reshape(1, -1))
  return out
